# Initial kernel scaffold; baseline (speedup 1.0000x reference)
import jax
import jax.numpy as jnp
from jax.experimental import pallas as pl


def kernel(x, prob_table):
    raise NotImplementedError("write your pallas kernel here")


# speedup vs baseline: 1.3762x; 1.3762x over previous
import functools

import jax
import jax.numpy as jnp
from jax import lax
from jax.experimental import pallas as pl
from jax.experimental.pallas import tpu as pltpu
from jax.experimental.pallas import tpu_sc as plsc

_NC = 2
_NS = 16
_NW = _NC * _NS
_CHUNK = 40


@functools.lru_cache(maxsize=None)
def _make_gather(bf: int, d: int):
    b_per_w = bf // _NW
    n_chunks = b_per_w // _CHUNK
    mesh = plsc.VectorSubcoreMesh(core_axis_name="c", subcore_axis_name="s")

    @functools.partial(
        pl.kernel,
        mesh=mesh,
        out_type=jax.ShapeDtypeStruct((bf, d), jnp.float32),
        scratch_types=[
            pltpu.VMEM((b_per_w,), jnp.int32),
            pltpu.VMEM((_CHUNK, d), jnp.float32),
            pltpu.SemaphoreType.DMA,
        ],
        compiler_params=pltpu.CompilerParams(use_tc_tiling_on_sc=False),
    )
    def gather_kernel(table_hbm, idx_hbm, out_hbm, idx_v, rows_v, sem):
        wid = lax.axis_index("s") * _NC + lax.axis_index("c")
        base = wid * b_per_w
        pltpu.sync_copy(idx_hbm.at[pl.ds(base, b_per_w)], idx_v)

        def body(g, carry):
            off = g * _CHUNK
            pltpu.async_copy(
                table_hbm.at[idx_v.at[pl.ds(off, _CHUNK)]], rows_v, sem
            ).wait()
            pltpu.sync_copy(rows_v, out_hbm.at[pl.ds(base + off, _CHUNK)])
            return carry

        lax.fori_loop(0, n_chunks, body, 0)

    return gather_kernel


def kernel(x, prob_table):
    b, l = x.shape
    v, d = prob_table.shape
    bf = b * l
    idx = x.reshape(bf).astype(jnp.int32)
    out = _make_gather(bf, d)(prob_table, idx)
    return out.reshape(b, l, d)


# speedup vs baseline: 1.4421x; 1.0478x over previous
import functools

import jax
import jax.numpy as jnp
from jax import lax
from jax.experimental import pallas as pl
from jax.experimental.pallas import tpu as pltpu
from jax.experimental.pallas import tpu_sc as plsc

_NC = 2
_NS = 16
_NW = _NC * _NS
_CHUNK = 64


@functools.lru_cache(maxsize=None)
def _make_gather(bf: int, d: int):
    b_per_w = bf // _NW
    n_chunks = b_per_w // _CHUNK
    mesh = plsc.VectorSubcoreMesh(core_axis_name="c", subcore_axis_name="s")

    @functools.partial(
        pl.kernel,
        mesh=mesh,
        out_type=jax.ShapeDtypeStruct((bf, d), jnp.float32),
        scratch_types=[
            pltpu.VMEM((b_per_w,), jnp.int32),
            pltpu.VMEM((_CHUNK, d), jnp.float32),
            pltpu.VMEM((_CHUNK, d), jnp.float32),
            pltpu.SemaphoreType.DMA,
            pltpu.SemaphoreType.DMA,
        ],
        compiler_params=pltpu.CompilerParams(use_tc_tiling_on_sc=False),
    )
    def gather_kernel(table_hbm, idx_hbm, out_hbm, idx_v, rows0, rows1, s0, s1):
        wid = lax.axis_index("s") * _NC + lax.axis_index("c")
        base = wid * b_per_w
        pltpu.sync_copy(idx_hbm.at[pl.ds(base, b_per_w)], idx_v)

        def gather(g, buf, sem):
            pltpu.async_copy(
                table_hbm.at[idx_v.at[pl.ds(g * _CHUNK, _CHUNK)]], buf, sem
            )

        def store(g, buf):
            pltpu.sync_copy(buf, out_hbm.at[pl.ds(base + g * _CHUNK, _CHUNK)])

        n_pairs = n_chunks // 2
        gather(0, rows0, s0)

        def body(h, carry):
            g = h * 2
            gather(g + 1, rows1, s1)
            pltpu.make_async_copy(
                table_hbm.at[idx_v.at[pl.ds(0, _CHUNK)]], rows0, s0
            ).wait()
            store(g, rows0)

            @pl.when(h < n_pairs - 1)
            def _():
                gather(g + 2, rows0, s0)

            pltpu.make_async_copy(
                table_hbm.at[idx_v.at[pl.ds(0, _CHUNK)]], rows1, s1
            ).wait()
            store(g + 1, rows1)
            return carry

        lax.fori_loop(0, n_pairs, body, 0)

    return gather_kernel


def kernel(x, prob_table):
    b, l = x.shape
    v, d = prob_table.shape
    bf = b * l
    idx = x.reshape(bf).astype(jnp.int32)
    out = _make_gather(bf, d)(prob_table, idx)
    return out.reshape(b, l, d)
